# baseline (speedup 1.0000x reference)
import functools

import jax
import jax.numpy as jnp
from jax import lax
from jax.experimental import pallas as pl
from jax.experimental.pallas import tpu as pltpu
from jax.experimental.pallas import tpu_sc as plsc

_CODE_DIM = 256
_NUM_CB = 4
_CB_SIZE = 1024
_SUB = _CODE_DIM // _NUM_CB

_ROWS = 2048
_SC_CHUNK = 1024


def _vq_kernel(z_ref, w_ref, wneg2_ref,
               idx_ref, loss_ref, perp_ref,
               loss_acc, c0, c1, c2, c3, *, n_rows, n_blocks):
    i = pl.program_id(0)
    counts = (c0, c1, c2, c3)

    @pl.when(i == 0)
    def _init():
        loss_acc[0] = 0.0
        for c in range(_NUM_CB):
            counts[c][...] = jnp.zeros((1, _CB_SIZE), jnp.float32)

    z = z_ref[...]
    loss_sum = loss_acc[0]
    lane_iota = jax.lax.broadcasted_iota(jnp.int32, (_ROWS, _CB_SIZE), 1)
    lane_iota_f = lane_iota.astype(jnp.float32)
    ones_row = jnp.ones((1, _ROWS), jnp.bfloat16)
    idx_parts = []
    for c in range(_NUM_CB):
        z_c = z[:, c * _SUB:(c + 1) * _SUB]
        w = w_ref[c]
        zz = jnp.sum(z_c * z_c, axis=1, keepdims=True)
        ww = jnp.sum(w * w, axis=1)[None, :]
        cross2 = jax.lax.dot_general(
            z_c, wneg2_ref[c], (((1,), (1,)), ((), ())),
            preferred_element_type=jnp.float32)
        dist = zz + cross2 + ww
        mind = jnp.min(dist, axis=1, keepdims=True)
        hit = dist == mind
        idxf = jnp.min(jnp.where(hit, lane_iota_f, 2048.0), axis=1)
        idx = idxf.astype(jnp.int32)
        onemask = lane_iota == idx[:, None]
        oh_b = onemask.astype(jnp.bfloat16)
        counts[c][...] += jax.lax.dot_general(
            ones_row, oh_b, (((1,), (0,)), ((), ())),
            preferred_element_type=jnp.float32)
        loss_sum = loss_sum + jnp.sum(mind)
        idx_parts.append(idx + (c * _CB_SIZE))
    idx_ref[...] = jnp.stack(idx_parts, axis=1)
    loss_acc[0] = loss_sum

    @pl.when(i == n_blocks - 1)
    def _finalize():
        loss_val = loss_acc[0] / float(_NUM_CB * n_rows * _SUB)
        loss_ref[...] = jnp.full((1, 1), loss_val, jnp.float32)
        perp = jnp.zeros((), jnp.float32)
        for c in range(_NUM_CB):
            p = counts[c][...] / float(n_rows)
            ent = -jnp.sum(p * jnp.log(jnp.clip(p, 1e-08)))
            perp = perp + jnp.exp(ent)
        perp_ref[...] = jnp.full((1, 1), perp / float(_NUM_CB), jnp.float32)


def _tc_call(zf, W, w_neg2, n, D, C, K, sub):
    n_blocks = n // _ROWS
    out_shapes = (
        jax.ShapeDtypeStruct((n, C), jnp.int32),
        jax.ShapeDtypeStruct((1, 1), jnp.float32),
        jax.ShapeDtypeStruct((1, 1), jnp.float32),
    )
    out_specs = (
        pl.BlockSpec((_ROWS, C), lambda i: (i, 0)),
        pl.BlockSpec((1, 1), lambda i: (0, 0)),
        pl.BlockSpec((1, 1), lambda i: (0, 0)),
    )
    in_specs = [
        pl.BlockSpec((_ROWS, D), lambda i: (i, 0)),
        pl.BlockSpec((C, K, sub), lambda i: (0, 0, 0)),
        pl.BlockSpec((C, K, sub), lambda i: (0, 0, 0)),
    ]
    scratch = [pltpu.SMEM((1,), jnp.float32)] + \
        [pltpu.VMEM((1, K), jnp.float32) for _ in range(C)]
    return pl.pallas_call(
        functools.partial(_vq_kernel, n_rows=n, n_blocks=n_blocks),
        grid=(n_blocks,),
        in_specs=in_specs,
        out_specs=out_specs,
        out_shape=out_shapes,
        scratch_shapes=scratch,
    )(zf, W, w_neg2)


def _sc_gather(w_flat, gidx_flat, m):
    n_workers = 32
    per_w = m // n_workers
    chunk = 1024
    chunk = 512
    n_chunks = per_w // chunk
    mesh = plsc.VectorSubcoreMesh(core_axis_name="c", subcore_axis_name="s")
    out_t = jax.ShapeDtypeStruct((m, 2 * _SUB), jnp.float32)

    @functools.partial(
        pl.kernel, mesh=mesh,
        out_type=(out_t, out_t),
        scratch_types=[
            pltpu.VMEM((per_w,), jnp.int32),
            pltpu.VMEM((chunk, 2 * _SUB), jnp.float32),
            pltpu.SemaphoreType.DMA,
        ],
    )
    def sc_kernel(w_hbm, idx_hbm, zq_hbm, st_hbm, idx_v, rows_v, sem):
        wid = lax.axis_index("s") * 2 + lax.axis_index("c")
        base = wid * per_w
        pltpu.sync_copy(idx_hbm.at[pl.ds(base, per_w)], idx_v)

        @pl.loop(0, n_chunks)
        def _(k):
            pltpu.async_copy(
                w_hbm.at[idx_v.at[pl.ds(k * chunk, chunk)]],
                rows_v, sem).wait()
            dst = pl.ds(base + k * chunk, chunk)
            pltpu.sync_copy(rows_v, zq_hbm.at[dst])
            pltpu.sync_copy(rows_v, st_hbm.at[dst])

    return sc_kernel(w_flat, gidx_flat)


@jax.jit
def kernel(z_e, W):
    B, L, D = z_e.shape
    C, K, sub = W.shape
    n = B * L
    zf = z_e.reshape(n, D)
    w_neg2 = W * (-2.0)

    gidx, loss, perp = _tc_call(zf, W, w_neg2, n, D, C, K, sub)

    w_pad = jnp.pad(W.reshape(C * K, sub), ((0, 0), (0, sub)))
    zq_w, st_w = _sc_gather(w_pad, gidx.reshape(n * C), n * C)

    offs = (jnp.arange(C, dtype=jnp.int32) * K)[None, :]
    indices_all = (gidx - offs).reshape(B, L, C)
    z_q_all = zq_w[:, :sub].reshape(B, L, D)
    z_q_st = st_w[:, :sub].reshape(B, L, D)
    loss_s = loss[0, 0]
    return (indices_all, z_q_st, z_q_all, loss_s, loss_s, perp[0, 0])


# speedup vs baseline: 1.1746x; 1.1746x over previous
import functools

import jax
import jax.numpy as jnp
from jax.experimental import pallas as pl
from jax.experimental.pallas import tpu as pltpu

_CODE_DIM = 256
_NUM_CB = 4
_CB_SIZE = 1024
_SUB = _CODE_DIM // _NUM_CB

_ROWS = 4096


def _vq_kernel(z_ref, w_ref, wneg2_ref, wb_ref,
               idx_ref, zq_ref, st_ref, loss_ref, perp_ref,
               loss_acc, c0, c1, c2, c3, *, n_rows, n_blocks):
    i = pl.program_id(0)
    counts = (c0, c1, c2, c3)

    @pl.when(i == 0)
    def _init():
        loss_acc[0] = 0.0
        for c in range(_NUM_CB):
            counts[c][...] = jnp.zeros((1, _CB_SIZE), jnp.float32)

    z = z_ref[...]
    loss_sum = loss_acc[0]
    lane_iota = jax.lax.broadcasted_iota(jnp.int32, (_ROWS, _CB_SIZE), 1)
    lane_iota_f = lane_iota.astype(jnp.float32)
    ones_row = jnp.ones((1, _ROWS), jnp.bfloat16)
    for c in range(_NUM_CB):
        z_c = z[:, c * _SUB:(c + 1) * _SUB]
        w = w_ref[c]
        zz = jnp.sum(z_c * z_c, axis=1, keepdims=True)
        ww = jnp.sum(w * w, axis=1)[None, :]
        cross2 = jax.lax.dot_general(
            z_c, wneg2_ref[c], (((1,), (1,)), ((), ())),
            preferred_element_type=jnp.float32)
        dist = zz + cross2 + ww
        mind = jnp.min(dist, axis=1, keepdims=True)
        hit = dist == mind
        idxf = jnp.min(jnp.where(hit, lane_iota_f, 2048.0), axis=1)
        idx = idxf.astype(jnp.int32)
        onemask = lane_iota == idx[:, None]
        oh_b = onemask.astype(jnp.bfloat16)
        zq_c = jax.lax.dot_general(
            oh_b, wb_ref[c], (((1,), (0,)), ((), ())),
            preferred_element_type=jnp.float32)
        counts[c][...] += jax.lax.dot_general(
            ones_row, oh_b, (((1,), (0,)), ((), ())),
            preferred_element_type=jnp.float32)
        loss_sum = loss_sum + jnp.sum(mind)
        idx_ref[0, c, 0, :] = idx
        zq_ref[:, c * _SUB:(c + 1) * _SUB] = zq_c
        st_ref[:, c * _SUB:(c + 1) * _SUB] = zq_c
    loss_acc[0] = loss_sum

    @pl.when(i == n_blocks - 1)
    def _finalize():
        loss_val = loss_acc[0] / float(_NUM_CB * n_rows * _SUB)
        loss_ref[...] = jnp.full((1, 1), loss_val, jnp.float32)
        perp = jnp.zeros((), jnp.float32)
        for c in range(_NUM_CB):
            p = counts[c][...] / float(n_rows)
            ent = -jnp.sum(p * jnp.log(jnp.clip(p, 1e-08)))
            perp = perp + jnp.exp(ent)
        perp_ref[...] = jnp.full((1, 1), perp / float(_NUM_CB), jnp.float32)


@jax.jit
def kernel(z_e, W):
    B, L, D = z_e.shape
    C, K, sub = W.shape
    n = B * L
    zf = z_e.reshape(n, D)
    n_blocks = n // _ROWS
    w_neg2 = W * (-2.0)
    w_b = W.astype(jnp.bfloat16)

    grid = (n_blocks,)
    out_shapes = (
        jax.ShapeDtypeStruct((n_blocks, C, 1, _ROWS), jnp.int32),
        jax.ShapeDtypeStruct((n, D), jnp.float32),
        jax.ShapeDtypeStruct((n, D), jnp.float32),
        jax.ShapeDtypeStruct((1, 1), jnp.float32),
        jax.ShapeDtypeStruct((1, 1), jnp.float32),
    )
    out_specs = (
        pl.BlockSpec((1, C, 1, _ROWS), lambda i: (i, 0, 0, 0)),
        pl.BlockSpec((_ROWS, D), lambda i: (i, 0)),
        pl.BlockSpec((_ROWS, D), lambda i: (i, 0)),
        pl.BlockSpec((1, 1), lambda i: (0, 0)),
        pl.BlockSpec((1, 1), lambda i: (0, 0)),
    )
    in_specs = [
        pl.BlockSpec((_ROWS, D), lambda i: (i, 0)),
        pl.BlockSpec((C, K, sub), lambda i: (0, 0, 0)),
        pl.BlockSpec((C, K, sub), lambda i: (0, 0, 0)),
        pl.BlockSpec((C, K, sub), lambda i: (0, 0, 0)),
    ]
    scratch = [pltpu.SMEM((1,), jnp.float32)] + \
        [pltpu.VMEM((1, K), jnp.float32) for _ in range(C)]

    idx_raw, zq, st, loss, perp = pl.pallas_call(
        functools.partial(_vq_kernel, n_rows=n, n_blocks=n_blocks),
        grid=grid,
        in_specs=in_specs,
        out_specs=out_specs,
        out_shape=out_shapes,
        scratch_shapes=scratch,
    )(zf, W, w_neg2, w_b)

    indices_all = idx_raw.reshape(n_blocks, C, _ROWS).transpose(0, 2, 1)
    indices_all = indices_all.reshape(B, L, C)
    z_q_all = zq.reshape(B, L, D)
    z_q_st = st.reshape(B, L, D)
    loss_s = loss[0, 0]
    return (indices_all, z_q_st, z_q_all, loss_s, loss_s, perp[0, 0])


# speedup vs baseline: 1.5952x; 1.3581x over previous
import functools

import jax
import jax.numpy as jnp
from jax.experimental import pallas as pl
from jax.experimental.pallas import tpu as pltpu

_CODE_DIM = 256
_NUM_CB = 4
_CB_SIZE = 1024
_SUB = _CODE_DIM // _NUM_CB

_ROWS = 2048


def _vq_kernel(z_ref, w_ref, wneg2_ref, wb_ref,
               idx_ref, zq_ref, st_ref, loss_ref, perp_ref,
               loss_acc, c0, c1, c2, c3, *, n_rows, n_blocks):
    i = pl.program_id(0)
    counts = (c0, c1, c2, c3)

    @pl.when(i == 0)
    def _init():
        loss_acc[0] = 0.0
        for c in range(_NUM_CB):
            counts[c][...] = jnp.zeros((1, _CB_SIZE), jnp.float32)

    z = z_ref[...]
    loss_sum = loss_acc[0]
    lane_iota = jax.lax.broadcasted_iota(jnp.int32, (_ROWS, _CB_SIZE), 1)
    lane_iota_f = lane_iota.astype(jnp.float32)
    ones_row = jnp.ones((1, _ROWS), jnp.bfloat16)
    for c in range(_NUM_CB):
        z_c = z[:, c * _SUB:(c + 1) * _SUB]
        w = w_ref[c]
        zz = jnp.sum(z_c * z_c, axis=1, keepdims=True)
        ww = jnp.sum(w * w, axis=1)[None, :]
        cross2 = jax.lax.dot_general(
            z_c, wneg2_ref[c], (((1,), (1,)), ((), ())),
            preferred_element_type=jnp.float32)
        dist = zz + cross2 + ww
        mind = jnp.min(dist, axis=1, keepdims=True)
        hit = dist == mind
        idxf = jnp.min(jnp.where(hit, lane_iota_f, 2048.0), axis=1)
        idx = idxf.astype(jnp.int32)
        onemask = lane_iota == idx[:, None]
        oh_b = onemask.astype(jnp.bfloat16)
        zq_c = jax.lax.dot_general(
            oh_b, wb_ref[c], (((1,), (0,)), ((), ())),
            preferred_element_type=jnp.float32)
        counts[c][...] += jax.lax.dot_general(
            ones_row, oh_b, (((1,), (0,)), ((), ())),
            preferred_element_type=jnp.float32)
        loss_sum = loss_sum + jnp.sum(mind)
        idx_ref[0, c, 0, :] = idx
        zq_ref[:, c * _SUB:(c + 1) * _SUB] = zq_c
        st_ref[:, c * _SUB:(c + 1) * _SUB] = zq_c
    loss_acc[0] = loss_sum

    @pl.when(i == n_blocks - 1)
    def _finalize():
        loss_val = loss_acc[0] / float(_NUM_CB * n_rows * _SUB)
        loss_ref[...] = jnp.full((1, 1), loss_val, jnp.float32)
        perp = jnp.zeros((), jnp.float32)
        for c in range(_NUM_CB):
            p = counts[c][...] / float(n_rows)
            ent = -jnp.sum(p * jnp.log(jnp.clip(p, 1e-08)))
            perp = perp + jnp.exp(ent)
        perp_ref[...] = jnp.full((1, 1), perp / float(_NUM_CB), jnp.float32)


@jax.jit
def kernel(z_e, W):
    B, L, D = z_e.shape
    C, K, sub = W.shape
    n = B * L
    zf = z_e.reshape(n, D)
    n_blocks = n // _ROWS
    w_neg2 = W * (-2.0)
    w_b = W.astype(jnp.bfloat16)

    grid = (n_blocks,)
    out_shapes = (
        jax.ShapeDtypeStruct((n_blocks, C, 1, _ROWS), jnp.int32),
        jax.ShapeDtypeStruct((n, D), jnp.float32),
        jax.ShapeDtypeStruct((n, D), jnp.float32),
        jax.ShapeDtypeStruct((1, 1), jnp.float32),
        jax.ShapeDtypeStruct((1, 1), jnp.float32),
    )
    out_specs = (
        pl.BlockSpec((1, C, 1, _ROWS), lambda i: (i, 0, 0, 0)),
        pl.BlockSpec((_ROWS, D), lambda i: (i, 0)),
        pl.BlockSpec((_ROWS, D), lambda i: (i, 0)),
        pl.BlockSpec((1, 1), lambda i: (0, 0)),
        pl.BlockSpec((1, 1), lambda i: (0, 0)),
    )
    in_specs = [
        pl.BlockSpec((_ROWS, D), lambda i: (i, 0)),
        pl.BlockSpec((C, K, sub), lambda i: (0, 0, 0)),
        pl.BlockSpec((C, K, sub), lambda i: (0, 0, 0)),
        pl.BlockSpec((C, K, sub), lambda i: (0, 0, 0)),
    ]
    scratch = [pltpu.SMEM((1,), jnp.float32)] + \
        [pltpu.VMEM((1, K), jnp.float32) for _ in range(C)]

    idx_raw, zq, st, loss, perp = pl.pallas_call(
        functools.partial(_vq_kernel, n_rows=n, n_blocks=n_blocks),
        grid=grid,
        in_specs=in_specs,
        out_specs=out_specs,
        out_shape=out_shapes,
        scratch_shapes=scratch,
    )(zf, W, w_neg2, w_b)

    indices_all = idx_raw.reshape(n_blocks, C, _ROWS).transpose(0, 2, 1)
    indices_all = indices_all.reshape(B, L, C)
    z_q_all = zq.reshape(B, L, D)
    z_q_st = st.reshape(B, L, D)
    loss_s = loss[0, 0]
    return (indices_all, z_q_st, z_q_all, loss_s, loss_s, perp[0, 0])
